# baseline (speedup 1.0000x reference)
import jax
import jax.numpy as jnp
from jax.experimental import pallas as pl
from jax.experimental.pallas import tpu as pltpu

_N = 2048
_C = 256
_K = 10
_BN = 128
_NB = _N // _BN
_BNK = _BN * _K
_BIG = 1e30
_INV_SQRT2 = 0.7071067811865476
_BF = jnp.bfloat16


def _mm_bf16(a, b):
    return jnp.dot(a.astype(_BF), b.astype(_BF),
                   preferred_element_type=jnp.float32)


def _mm_exact(a, b):
    return jnp.dot(a, b, precision=jax.lax.Precision.HIGHEST,
                   preferred_element_type=jnp.float32)


def _body(q_ref, qpos_ref, vposT_ref, Wv_ref, bv_ref, W1a_ref, W1b_ref,
          b1_ref, lng_ref, lnb_ref, W2_ref, Wka_ref, Wkb_ref, bk_ref,
          out_ref, voff_ref, h2_ref, t2_ref):
    i = pl.program_id(0)

    @pl.when(i == 0)
    def _init():
        qq = q_ref[...]
        voff_ref[...] = _mm_bf16(qq, Wv_ref[...]) + bv_ref[...]
        h2_ref[...] = _mm_bf16(qq, W1b_ref[...]) + b1_ref[...]
        t2_ref[...] = _mm_bf16(qq, Wkb_ref[...]) + bk_ref[...]

    v0 = vposT_ref[0:1, :]
    v1 = vposT_ref[1:2, :]
    v2 = vposT_ref[2:3, :]
    vn2 = v0 * v0 + v1 * v1 + v2 * v2

    qb = qpos_ref[pl.ds(i * _BN, _BN), :]
    q0 = qb[:, 0:1]
    q1 = qb[:, 1:2]
    q2 = qb[:, 2:3]
    qn2 = q0 * q0 + q1 * q1 + q2 * q2

    d2 = -2.0 * _mm_bf16(qb, vposT_ref[...]) + qn2 + vn2

    iota_b = jax.lax.broadcasted_iota(jnp.int32, (_BN, _N), 1).astype(
        jnp.float32)
    masks = []
    p0s, p1s, p2s = [], [], []
    for _ in range(_K):
        m = jnp.min(d2, axis=1, keepdims=True)
        cand = jnp.where(d2 == m, iota_b, float(_N))
        amin = jnp.min(cand, axis=1, keepdims=True)
        msk = iota_b == amin
        mf = msk.astype(jnp.float32)
        masks.append(mf)
        p0s.append(jnp.sum(mf * v0, axis=1, keepdims=True))
        p1s.append(jnp.sum(mf * v1, axis=1, keepdims=True))
        p2s.append(jnp.sum(mf * v2, axis=1, keepdims=True))
        d2 = jnp.where(msk, _BIG, d2)

    def _minmax(ps):
        lo, hi = ps[0], ps[0]
        for p in ps[1:]:
            lo = jnp.minimum(lo, p)
            hi = jnp.maximum(hi, p)
        return (hi - lo) * 0.5

    sc0 = _minmax(p0s)
    sc1 = _minmax(p1s)
    sc2 = _minmax(p2s)

    G = jnp.concatenate(masks, axis=0)
    lp0 = jnp.concatenate(p0s, axis=0)
    lp1 = jnp.concatenate(p1s, axis=0)
    lp2 = jnp.concatenate(p2s, axis=0)

    offl = _mm_exact(G, voff_ref[...])
    h2b = h2_ref[pl.ds(i * _BN, _BN), :]
    h = _mm_bf16(offl, W1a_ref[...]) + jnp.concatenate([h2b] * _K, axis=0)

    mean = jnp.mean(h, axis=1, keepdims=True)
    hc = h - mean
    var = jnp.mean(hc * hc, axis=1, keepdims=True)
    hn = hc / jnp.sqrt(var + 1e-5) * lng_ref[...] + lnb_ref[...]
    ge = 0.5 * hn * (1.0 + jax.lax.erf(hn * _INV_SQRT2))
    offs = jnp.tanh(_mm_bf16(ge, W2_ref[...]))

    s0 = lp0 + offs[:, 0:1] * jnp.concatenate([sc0] * _K, axis=0)
    s1 = lp1 + offs[:, 1:2] * jnp.concatenate([sc1] * _K, axis=0)
    s2 = lp2 + offs[:, 2:3] * jnp.concatenate([sc2] * _K, axis=0)
    sn2 = s0 * s0 + s1 * s1 + s2 * s2
    shift = jnp.concatenate([s0, s1, s2], axis=1)

    d2b = -2.0 * _mm_bf16(shift, vposT_ref[...]) + sn2 + vn2

    iota_g = jax.lax.broadcasted_iota(jnp.int32, (_BNK, _N), 1).astype(
        jnp.float32)
    Wacc = jnp.zeros((_BNK, _N), jnp.float32)
    R = jnp.zeros((_BNK, 1), jnp.float32)
    for _ in range(3):
        m = jnp.min(d2b, axis=1, keepdims=True)
        cand = jnp.where(d2b == m, iota_g, float(_N))
        amin = jnp.min(cand, axis=1, keepdims=True)
        msk = iota_g == amin
        dist = jnp.sqrt(jnp.clip(m, 1e-12, None))
        r = 1.0 / (dist + 1e-8)
        Wacc = Wacc + msk.astype(jnp.float32) * r
        R = R + r
        d2b = jnp.where(msk, _BIG, d2b)
    W3 = Wacc / R

    interp = _mm_exact(W3, q_ref[...])
    qblk = q_ref[pl.ds(i * _BN, _BN), :]
    f = interp - jnp.concatenate([qblk] * _K, axis=0)
    t2b = t2_ref[pl.ds(i * _BN, _BN), :]
    o = _mm_bf16(f, Wka_ref[...]) + jnp.concatenate([t2b] * _K, axis=0)
    o = jnp.where(o >= 0, o, 0.2 * o)

    acc = o[0:_BN, :]
    for k in range(1, _K):
        acc = jnp.maximum(acc, o[k * _BN:(k + 1) * _BN, :])
    out_ref[...] = acc


def kernel(q, q_pos, Wv, bv, W1, b1, ln_g, ln_b, W2, Wk, bk):
    B, N, C = q.shape
    q2 = q[0]
    qpos = q_pos[0]
    vposT = qpos.T
    W1a, W1b = W1[:C], W1[C:]
    Wka = Wk[:C]
    Wkb = Wk[C:]

    full = lambda shape: pl.BlockSpec(shape, lambda i: (0,) * len(shape))
    out = pl.pallas_call(
        _body,
        grid=(_NB,),
        in_specs=[
            full((_N, _C)),
            full((_N, 3)),
            full((3, _N)),
            full((_C, _C)),
            full((1, _C)),
            full((_C, _C)),
            full((_C, _C)),
            full((1, _C)),
            full((1, _C)),
            full((1, _C)),
            full((_C, 3)),
            full((_C, _C)),
            full((_C, _C)),
            full((1, _C)),
        ],
        out_specs=pl.BlockSpec((_BN, _C), lambda i: (i, 0)),
        out_shape=jax.ShapeDtypeStruct((_N, _C), jnp.float32),
        scratch_shapes=[
            pltpu.VMEM((_N, _C), jnp.float32),
            pltpu.VMEM((_N, _C), jnp.float32),
            pltpu.VMEM((_N, _C), jnp.float32),
        ],
        compiler_params=pltpu.CompilerParams(
            dimension_semantics=("arbitrary",)),
    )(q2, qpos, vposT, Wv, bv[None, :], W1a, W1b, b1[None, :],
      ln_g[None, :], ln_b[None, :], W2, Wka, Wkb, bk[None, :])
    return out[None]


# speedup vs baseline: 1.0795x; 1.0795x over previous
import jax
import jax.numpy as jnp
from jax.experimental import pallas as pl
from jax.experimental.pallas import tpu as pltpu

_N = 2048
_C = 256
_K = 10
_BN = 32
_NB = _N // _BN
_BNK = _BN * _K
_BIG = 1e30
_INV_SQRT2 = 0.7071067811865476
_BF = jnp.bfloat16


def _mm_bf16(a, b):
    return jnp.dot(a.astype(_BF), b.astype(_BF),
                   preferred_element_type=jnp.float32)


def _mm_exact(a, b):
    return jnp.dot(a, b, precision=jax.lax.Precision.HIGHEST,
                   preferred_element_type=jnp.float32)


def _body(q_ref, qpos_ref, vposT_ref, Wv_ref, bv_ref, W1a_ref, W1b_ref,
          b1_ref, lng_ref, lnb_ref, W2_ref, Wka_ref, Wkb_ref, bk_ref,
          out_ref, voff_ref, h2_ref, t2_ref):
    i = pl.program_id(0)

    @pl.when(i == 0)
    def _init():
        qq = q_ref[...]
        voff_ref[...] = _mm_bf16(qq, Wv_ref[...]) + bv_ref[...]
        h2_ref[...] = _mm_bf16(qq, W1b_ref[...]) + b1_ref[...]
        t2_ref[...] = _mm_bf16(qq, Wkb_ref[...]) + bk_ref[...]

    v0 = vposT_ref[0:1, :]
    v1 = vposT_ref[1:2, :]
    v2 = vposT_ref[2:3, :]
    vn2 = v0 * v0 + v1 * v1 + v2 * v2

    qb = qpos_ref[pl.ds(i * _BN, _BN), :]
    q0 = qb[:, 0:1]
    q1 = qb[:, 1:2]
    q2 = qb[:, 2:3]
    qn2 = q0 * q0 + q1 * q1 + q2 * q2

    d2 = -2.0 * _mm_bf16(qb, vposT_ref[...]) + qn2 + vn2

    iota_b = jax.lax.broadcasted_iota(jnp.int32, (_BN, _N), 1).astype(
        jnp.float32)
    masks = []
    p0s, p1s, p2s = [], [], []
    for _ in range(_K):
        m = jnp.min(d2, axis=1, keepdims=True)
        cand = jnp.where(d2 == m, iota_b, float(_N))
        amin = jnp.min(cand, axis=1, keepdims=True)
        msk = iota_b == amin
        mf = msk.astype(jnp.float32)
        masks.append(mf)
        p0s.append(jnp.sum(mf * v0, axis=1, keepdims=True))
        p1s.append(jnp.sum(mf * v1, axis=1, keepdims=True))
        p2s.append(jnp.sum(mf * v2, axis=1, keepdims=True))
        d2 = jnp.where(msk, _BIG, d2)

    def _minmax(ps):
        lo, hi = ps[0], ps[0]
        for p in ps[1:]:
            lo = jnp.minimum(lo, p)
            hi = jnp.maximum(hi, p)
        return (hi - lo) * 0.5

    sc0 = _minmax(p0s)
    sc1 = _minmax(p1s)
    sc2 = _minmax(p2s)

    G = jnp.concatenate(masks, axis=0)
    lp0 = jnp.concatenate(p0s, axis=0)
    lp1 = jnp.concatenate(p1s, axis=0)
    lp2 = jnp.concatenate(p2s, axis=0)

    offl = _mm_exact(G, voff_ref[...])
    h2b = h2_ref[pl.ds(i * _BN, _BN), :]
    h = _mm_bf16(offl, W1a_ref[...]) + jnp.concatenate([h2b] * _K, axis=0)

    mean = jnp.mean(h, axis=1, keepdims=True)
    hc = h - mean
    var = jnp.mean(hc * hc, axis=1, keepdims=True)
    hn = hc / jnp.sqrt(var + 1e-5) * lng_ref[...] + lnb_ref[...]
    ge = 0.5 * hn * (1.0 + jax.lax.erf(hn * _INV_SQRT2))
    offs = jnp.tanh(_mm_bf16(ge, W2_ref[...]))

    s0 = lp0 + offs[:, 0:1] * jnp.concatenate([sc0] * _K, axis=0)
    s1 = lp1 + offs[:, 1:2] * jnp.concatenate([sc1] * _K, axis=0)
    s2 = lp2 + offs[:, 2:3] * jnp.concatenate([sc2] * _K, axis=0)
    sn2 = s0 * s0 + s1 * s1 + s2 * s2
    shift = jnp.concatenate([s0, s1, s2], axis=1)

    d2b = -2.0 * _mm_bf16(shift, vposT_ref[...]) + sn2 + vn2

    iota_g = jax.lax.broadcasted_iota(jnp.int32, (_BNK, _N), 1).astype(
        jnp.float32)
    Wacc = jnp.zeros((_BNK, _N), jnp.float32)
    R = jnp.zeros((_BNK, 1), jnp.float32)
    for _ in range(3):
        m = jnp.min(d2b, axis=1, keepdims=True)
        cand = jnp.where(d2b == m, iota_g, float(_N))
        amin = jnp.min(cand, axis=1, keepdims=True)
        msk = iota_g == amin
        dist = jnp.sqrt(jnp.clip(m, 1e-12, None))
        r = 1.0 / (dist + 1e-8)
        Wacc = Wacc + msk.astype(jnp.float32) * r
        R = R + r
        d2b = jnp.where(msk, _BIG, d2b)
    W3 = Wacc / R

    interp = _mm_exact(W3, q_ref[...])
    qblk = q_ref[pl.ds(i * _BN, _BN), :]
    f = interp - jnp.concatenate([qblk] * _K, axis=0)
    t2b = t2_ref[pl.ds(i * _BN, _BN), :]
    o = _mm_bf16(f, Wka_ref[...]) + jnp.concatenate([t2b] * _K, axis=0)
    o = jnp.where(o >= 0, o, 0.2 * o)

    acc = o[0:_BN, :]
    for k in range(1, _K):
        acc = jnp.maximum(acc, o[k * _BN:(k + 1) * _BN, :])
    out_ref[...] = acc


def kernel(q, q_pos, Wv, bv, W1, b1, ln_g, ln_b, W2, Wk, bk):
    B, N, C = q.shape
    q2 = q[0]
    qpos = q_pos[0]
    vposT = qpos.T
    W1a, W1b = W1[:C], W1[C:]
    Wka = Wk[:C]
    Wkb = Wk[C:]

    full = lambda shape: pl.BlockSpec(shape, lambda i: (0,) * len(shape))
    out = pl.pallas_call(
        _body,
        grid=(_NB,),
        in_specs=[
            full((_N, _C)),
            full((_N, 3)),
            full((3, _N)),
            full((_C, _C)),
            full((1, _C)),
            full((_C, _C)),
            full((_C, _C)),
            full((1, _C)),
            full((1, _C)),
            full((1, _C)),
            full((_C, 3)),
            full((_C, _C)),
            full((_C, _C)),
            full((1, _C)),
        ],
        out_specs=pl.BlockSpec((_BN, _C), lambda i: (i, 0)),
        out_shape=jax.ShapeDtypeStruct((_N, _C), jnp.float32),
        scratch_shapes=[
            pltpu.VMEM((_N, _C), jnp.float32),
            pltpu.VMEM((_N, _C), jnp.float32),
            pltpu.VMEM((_N, _C), jnp.float32),
        ],
        compiler_params=pltpu.CompilerParams(
            dimension_semantics=("arbitrary",)),
    )(q2, qpos, vposT, Wv, bv[None, :], W1a, W1b, b1[None, :],
      ln_g[None, :], ln_b[None, :], W2, Wka, Wkb, bk[None, :])
    return out[None]


# speedup vs baseline: 1.2092x; 1.1201x over previous
import jax
import jax.numpy as jnp
from jax.experimental import pallas as pl
from jax.experimental.pallas import tpu as pltpu

_N = 2048
_C = 256
_K = 10
_BN = 64
_NB = _N // _BN
_BNK = _BN * _K
_BIG = 1e30
_INV_SQRT2 = 0.7071067811865476
_BF = jnp.bfloat16


def _mm_bf16(a, b):
    return jnp.dot(a.astype(_BF), b.astype(_BF),
                   preferred_element_type=jnp.float32)


def _mm_exact(a, b):
    return jnp.dot(a, b, precision=jax.lax.Precision.HIGHEST,
                   preferred_element_type=jnp.float32)


def _body(q_ref, qpos_ref, vposT_ref, Wv_ref, bv_ref, W1a_ref, W1b_ref,
          b1_ref, lng_ref, lnb_ref, W2_ref, Wka_ref, Wkb_ref, bk_ref,
          out_ref, voff_ref, h2_ref, t2_ref):
    i = pl.program_id(0)

    @pl.when(i == 0)
    def _init():
        qq = q_ref[...]
        voff_ref[...] = _mm_bf16(qq, Wv_ref[...]) + bv_ref[...]
        h2_ref[...] = _mm_bf16(qq, W1b_ref[...]) + b1_ref[...]
        t2_ref[...] = _mm_bf16(qq, Wkb_ref[...]) + bk_ref[...]

    v0 = vposT_ref[0:1, :]
    v1 = vposT_ref[1:2, :]
    v2 = vposT_ref[2:3, :]
    vn2 = v0 * v0 + v1 * v1 + v2 * v2

    qb = qpos_ref[pl.ds(i * _BN, _BN), :]
    q0 = qb[:, 0:1]
    q1 = qb[:, 1:2]
    q2 = qb[:, 2:3]
    qn2 = q0 * q0 + q1 * q1 + q2 * q2

    d2 = -2.0 * _mm_bf16(qb, vposT_ref[...]) + qn2 + vn2

    iota_b = jax.lax.broadcasted_iota(jnp.int32, (_BN, _N), 1).astype(
        jnp.float32)
    masks = []
    p0s, p1s, p2s = [], [], []
    for _ in range(_K):
        m = jnp.min(d2, axis=1, keepdims=True)
        cand = jnp.where(d2 == m, iota_b, float(_N))
        amin = jnp.min(cand, axis=1, keepdims=True)
        msk = iota_b == amin
        mf = msk.astype(jnp.float32)
        masks.append(mf)
        p0s.append(jnp.sum(mf * v0, axis=1, keepdims=True))
        p1s.append(jnp.sum(mf * v1, axis=1, keepdims=True))
        p2s.append(jnp.sum(mf * v2, axis=1, keepdims=True))
        d2 = jnp.where(msk, _BIG, d2)

    def _minmax(ps):
        lo, hi = ps[0], ps[0]
        for p in ps[1:]:
            lo = jnp.minimum(lo, p)
            hi = jnp.maximum(hi, p)
        return (hi - lo) * 0.5

    sc0 = _minmax(p0s)
    sc1 = _minmax(p1s)
    sc2 = _minmax(p2s)

    G = jnp.concatenate(masks, axis=0)
    lp0 = jnp.concatenate(p0s, axis=0)
    lp1 = jnp.concatenate(p1s, axis=0)
    lp2 = jnp.concatenate(p2s, axis=0)

    offl = _mm_exact(G, voff_ref[...])
    h2b = h2_ref[pl.ds(i * _BN, _BN), :]
    h = _mm_bf16(offl, W1a_ref[...]) + jnp.concatenate([h2b] * _K, axis=0)

    mean = jnp.mean(h, axis=1, keepdims=True)
    hc = h - mean
    var = jnp.mean(hc * hc, axis=1, keepdims=True)
    hn = hc / jnp.sqrt(var + 1e-5) * lng_ref[...] + lnb_ref[...]
    ge = 0.5 * hn * (1.0 + jax.lax.erf(hn * _INV_SQRT2))
    offs = jnp.tanh(_mm_bf16(ge, W2_ref[...]))

    s0 = lp0 + offs[:, 0:1] * jnp.concatenate([sc0] * _K, axis=0)
    s1 = lp1 + offs[:, 1:2] * jnp.concatenate([sc1] * _K, axis=0)
    s2 = lp2 + offs[:, 2:3] * jnp.concatenate([sc2] * _K, axis=0)
    sn2 = s0 * s0 + s1 * s1 + s2 * s2
    shift = jnp.concatenate([s0, s1, s2], axis=1)

    d2b = -2.0 * _mm_bf16(shift, vposT_ref[...]) + sn2 + vn2

    iota_g = jax.lax.broadcasted_iota(jnp.int32, (_BNK, _N), 1).astype(
        jnp.float32)
    Wacc = jnp.zeros((_BNK, _N), jnp.float32)
    R = jnp.zeros((_BNK, 1), jnp.float32)
    for _ in range(3):
        m = jnp.min(d2b, axis=1, keepdims=True)
        cand = jnp.where(d2b == m, iota_g, float(_N))
        amin = jnp.min(cand, axis=1, keepdims=True)
        msk = iota_g == amin
        dist = jnp.sqrt(jnp.clip(m, 1e-12, None))
        r = 1.0 / (dist + 1e-8)
        Wacc = Wacc + msk.astype(jnp.float32) * r
        R = R + r
        d2b = jnp.where(msk, _BIG, d2b)
    W3 = Wacc / R

    interp = _mm_exact(W3, q_ref[...])
    qblk = q_ref[pl.ds(i * _BN, _BN), :]
    f = interp - jnp.concatenate([qblk] * _K, axis=0)
    t2b = t2_ref[pl.ds(i * _BN, _BN), :]
    o = _mm_bf16(f, Wka_ref[...]) + jnp.concatenate([t2b] * _K, axis=0)
    o = jnp.where(o >= 0, o, 0.2 * o)

    acc = o[0:_BN, :]
    for k in range(1, _K):
        acc = jnp.maximum(acc, o[k * _BN:(k + 1) * _BN, :])
    out_ref[...] = acc


def kernel(q, q_pos, Wv, bv, W1, b1, ln_g, ln_b, W2, Wk, bk):
    B, N, C = q.shape
    q2 = q[0]
    qpos = q_pos[0]
    vposT = qpos.T
    W1a, W1b = W1[:C], W1[C:]
    Wka = Wk[:C]
    Wkb = Wk[C:]

    full = lambda shape: pl.BlockSpec(shape, lambda i: (0,) * len(shape))
    out = pl.pallas_call(
        _body,
        grid=(_NB,),
        in_specs=[
            full((_N, _C)),
            full((_N, 3)),
            full((3, _N)),
            full((_C, _C)),
            full((1, _C)),
            full((_C, _C)),
            full((_C, _C)),
            full((1, _C)),
            full((1, _C)),
            full((1, _C)),
            full((_C, 3)),
            full((_C, _C)),
            full((_C, _C)),
            full((1, _C)),
        ],
        out_specs=pl.BlockSpec((_BN, _C), lambda i: (i, 0)),
        out_shape=jax.ShapeDtypeStruct((_N, _C), jnp.float32),
        scratch_shapes=[
            pltpu.VMEM((_N, _C), jnp.float32),
            pltpu.VMEM((_N, _C), jnp.float32),
            pltpu.VMEM((_N, _C), jnp.float32),
        ],
        compiler_params=pltpu.CompilerParams(
            dimension_semantics=("arbitrary",)),
    )(q2, qpos, vposT, Wv, bv[None, :], W1a, W1b, b1[None, :],
      ln_g[None, :], ln_b[None, :], W2, Wka, Wkb, bk[None, :])
    return out[None]


# speedup vs baseline: 1.7810x; 1.4729x over previous
import jax
import jax.numpy as jnp
from jax.experimental import pallas as pl
from jax.experimental.pallas import tpu as pltpu

_N = 2048
_C = 256
_K = 10
_BN = 64
_NB = _N // _BN
_BNK = _BN * _K
_BIG = 1e30
_INV_SQRT2 = 0.7071067811865476
_BF = jnp.bfloat16


def _mm_bf16(a, b):
    return jnp.dot(a.astype(_BF), b.astype(_BF),
                   preferred_element_type=jnp.float32)


def _mm_bf(a_bf, b_bf):
    return jnp.dot(a_bf, b_bf, preferred_element_type=jnp.float32)


def _split3(x):
    hi = x.astype(_BF)
    r1 = x - hi.astype(jnp.float32)
    mid = r1.astype(_BF)
    r2 = r1 - mid.astype(jnp.float32)
    return hi, mid, r2.astype(_BF)


def _body(q_ref, qpos_ref, vposT_ref, Wv_ref, bv_ref, W1a_ref, W1b_ref,
          b1_ref, lng_ref, lnb_ref, W2_ref, Wka_ref, Wkb_ref, bk_ref,
          out_ref, vh_ref, vm_ref, vl_ref, qh_ref, qm_ref, ql_ref,
          h2_ref, t2_ref):
    i = pl.program_id(0)

    @pl.when(i == 0)
    def _init():
        qq = q_ref[...]
        voff = _mm_bf16(qq, Wv_ref[...]) + bv_ref[...]
        vh_ref[...], vm_ref[...], vl_ref[...] = _split3(voff)
        qh_ref[...], qm_ref[...], ql_ref[...] = _split3(qq)
        h2_ref[...] = _mm_bf16(qq, W1b_ref[...]) + b1_ref[...]
        t2_ref[...] = _mm_bf16(qq, Wkb_ref[...]) + bk_ref[...]

    v0 = vposT_ref[0:1, :]
    v1 = vposT_ref[1:2, :]
    v2 = vposT_ref[2:3, :]
    vn2 = v0 * v0 + v1 * v1 + v2 * v2

    qb = qpos_ref[pl.ds(i * _BN, _BN), :]
    q0 = qb[:, 0:1]
    q1 = qb[:, 1:2]
    q2 = qb[:, 2:3]
    qn2 = q0 * q0 + q1 * q1 + q2 * q2

    d2 = -2.0 * _mm_bf16(qb, vposT_ref[...]) + qn2 + vn2

    iota_b = jax.lax.broadcasted_iota(jnp.int32, (_BN, _N), 1).astype(
        jnp.float32)
    masks = []
    p0s, p1s, p2s = [], [], []
    for _ in range(_K):
        m = jnp.min(d2, axis=1, keepdims=True)
        cand = jnp.where(d2 == m, iota_b, float(_N))
        amin = jnp.min(cand, axis=1, keepdims=True)
        msk = iota_b == amin
        mf = msk.astype(jnp.float32)
        masks.append(mf)
        p0s.append(jnp.sum(mf * v0, axis=1, keepdims=True))
        p1s.append(jnp.sum(mf * v1, axis=1, keepdims=True))
        p2s.append(jnp.sum(mf * v2, axis=1, keepdims=True))
        d2 = jnp.where(msk, _BIG, d2)

    def _minmax(ps):
        lo, hi = ps[0], ps[0]
        for p in ps[1:]:
            lo = jnp.minimum(lo, p)
            hi = jnp.maximum(hi, p)
        return (hi - lo) * 0.5

    sc0 = _minmax(p0s)
    sc1 = _minmax(p1s)
    sc2 = _minmax(p2s)

    G = jnp.concatenate(masks, axis=0)
    lp0 = jnp.concatenate(p0s, axis=0)
    lp1 = jnp.concatenate(p1s, axis=0)
    lp2 = jnp.concatenate(p2s, axis=0)

    Gb = G.astype(_BF)
    offl = (_mm_bf(Gb, vh_ref[...]) + _mm_bf(Gb, vm_ref[...])
            + _mm_bf(Gb, vl_ref[...]))
    h2b = h2_ref[pl.ds(i * _BN, _BN), :]
    h = _mm_bf16(offl, W1a_ref[...]) + jnp.concatenate([h2b] * _K, axis=0)

    mean = jnp.mean(h, axis=1, keepdims=True)
    hc = h - mean
    var = jnp.mean(hc * hc, axis=1, keepdims=True)
    hn = hc / jnp.sqrt(var + 1e-5) * lng_ref[...] + lnb_ref[...]
    ge = 0.5 * hn * (1.0 + jax.lax.erf(hn * _INV_SQRT2))
    offs = jnp.tanh(_mm_bf16(ge, W2_ref[...]))

    s0 = lp0 + offs[:, 0:1] * jnp.concatenate([sc0] * _K, axis=0)
    s1 = lp1 + offs[:, 1:2] * jnp.concatenate([sc1] * _K, axis=0)
    s2 = lp2 + offs[:, 2:3] * jnp.concatenate([sc2] * _K, axis=0)
    sn2 = s0 * s0 + s1 * s1 + s2 * s2
    shift = jnp.concatenate([s0, s1, s2], axis=1)

    d2b = -2.0 * _mm_bf16(shift, vposT_ref[...]) + sn2 + vn2

    iota_g = jax.lax.broadcasted_iota(jnp.int32, (_BNK, _N), 1).astype(
        jnp.float32)
    Wacc = jnp.zeros((_BNK, _N), jnp.float32)
    R = jnp.zeros((_BNK, 1), jnp.float32)
    for _ in range(3):
        m = jnp.min(d2b, axis=1, keepdims=True)
        cand = jnp.where(d2b == m, iota_g, float(_N))
        amin = jnp.min(cand, axis=1, keepdims=True)
        msk = iota_g == amin
        dist = jnp.sqrt(jnp.clip(m, 1e-12, None))
        r = 1.0 / (dist + 1e-8)
        Wacc = Wacc + msk.astype(jnp.float32) * r
        R = R + r
        d2b = jnp.where(msk, _BIG, d2b)
    W3 = Wacc / R

    W3h = W3.astype(_BF)
    W3l = (W3 - W3h.astype(jnp.float32)).astype(_BF)
    interp = (_mm_bf(W3h, qh_ref[...]) + _mm_bf(W3h, qm_ref[...])
              + _mm_bf(W3l, qh_ref[...]))
    qblk = q_ref[pl.ds(i * _BN, _BN), :]
    f = interp - jnp.concatenate([qblk] * _K, axis=0)
    t2b = t2_ref[pl.ds(i * _BN, _BN), :]
    o = _mm_bf16(f, Wka_ref[...]) + jnp.concatenate([t2b] * _K, axis=0)
    o = jnp.where(o >= 0, o, 0.2 * o)

    acc = o[0:_BN, :]
    for k in range(1, _K):
        acc = jnp.maximum(acc, o[k * _BN:(k + 1) * _BN, :])
    out_ref[...] = acc


def kernel(q, q_pos, Wv, bv, W1, b1, ln_g, ln_b, W2, Wk, bk):
    B, N, C = q.shape
    q2 = q[0]
    qpos = q_pos[0]
    vposT = qpos.T
    W1a, W1b = W1[:C], W1[C:]
    Wka = Wk[:C]
    Wkb = Wk[C:]

    full = lambda shape: pl.BlockSpec(shape, lambda i: (0,) * len(shape))
    out = pl.pallas_call(
        _body,
        grid=(_NB,),
        in_specs=[
            full((_N, _C)),
            full((_N, 3)),
            full((3, _N)),
            full((_C, _C)),
            full((1, _C)),
            full((_C, _C)),
            full((_C, _C)),
            full((1, _C)),
            full((1, _C)),
            full((1, _C)),
            full((_C, 3)),
            full((_C, _C)),
            full((_C, _C)),
            full((1, _C)),
        ],
        out_specs=pl.BlockSpec((_BN, _C), lambda i: (i, 0)),
        out_shape=jax.ShapeDtypeStruct((_N, _C), jnp.float32),
        scratch_shapes=[
            pltpu.VMEM((_N, _C), jnp.bfloat16),
            pltpu.VMEM((_N, _C), jnp.bfloat16),
            pltpu.VMEM((_N, _C), jnp.bfloat16),
            pltpu.VMEM((_N, _C), jnp.bfloat16),
            pltpu.VMEM((_N, _C), jnp.bfloat16),
            pltpu.VMEM((_N, _C), jnp.bfloat16),
            pltpu.VMEM((_N, _C), jnp.float32),
            pltpu.VMEM((_N, _C), jnp.float32),
        ],
        compiler_params=pltpu.CompilerParams(
            dimension_semantics=("arbitrary",)),
    )(q2, qpos, vposT, Wv, bv[None, :], W1a, W1b, b1[None, :],
      ln_g[None, :], ln_b[None, :], W2, Wka, Wkb, bk[None, :])
    return out[None]


# speedup vs baseline: 1.8717x; 1.0509x over previous
import jax
import jax.numpy as jnp
from jax.experimental import pallas as pl
from jax.experimental.pallas import tpu as pltpu

_N = 2048
_C = 256
_K = 10
_BN = 64
_NB = _N // _BN
_BNK = _BN * _K
_BIG = 1e30
_INV_SQRT2 = 0.7071067811865476
_BF = jnp.bfloat16


def _mm_bf16(a, b):
    return jnp.dot(a.astype(_BF), b.astype(_BF),
                   preferred_element_type=jnp.float32)


def _mm_bf(a_bf, b_bf):
    return jnp.dot(a_bf, b_bf, preferred_element_type=jnp.float32)


def _split3(x):
    hi = x.astype(_BF)
    r1 = x - hi.astype(jnp.float32)
    mid = r1.astype(_BF)
    r2 = r1 - mid.astype(jnp.float32)
    return hi, mid, r2.astype(_BF)


def _gather3(g_bf, h_ref, m_ref, l_ref):
    return (_mm_bf(g_bf, h_ref[...]) + _mm_bf(g_bf, m_ref[...])
            + _mm_bf(g_bf, l_ref[...]))


def _body(q_ref, qpos_ref, vposT_ref, ph_ref, pm_ref, pl_ref,
          Wv_ref, bv_ref, W1a_ref, W1b_ref,
          b1_ref, lng_ref, lnb_ref, W2_ref, Wka_ref, Wkb_ref, bk_ref,
          out_ref, vh_ref, vm_ref, vl_ref, qh_ref, qm_ref,
          h2_ref, t2_ref):
    i = pl.program_id(0)

    @pl.when(i == 0)
    def _init():
        qq = q_ref[...]
        voff = _mm_bf16(qq, Wv_ref[...]) + bv_ref[...]
        vh_ref[...], vm_ref[...], vl_ref[...] = _split3(voff)
        qh, qm, _ = _split3(qq)
        qh_ref[...] = qh
        qm_ref[...] = qm
        h2_ref[...] = _mm_bf16(qq, W1b_ref[...]) + b1_ref[...]
        t2_ref[...] = _mm_bf16(qq, Wkb_ref[...]) + bk_ref[...]

    v0 = vposT_ref[0:1, :]
    v1 = vposT_ref[1:2, :]
    v2 = vposT_ref[2:3, :]
    vn2 = v0 * v0 + v1 * v1 + v2 * v2

    qb = qpos_ref[pl.ds(i * _BN, _BN), :]

    q0 = qb[:, 0:1]
    q1 = qb[:, 1:2]
    q2 = qb[:, 2:3]
    qn2 = q0 * q0 + q1 * q1 + q2 * q2
    e = -2.0 * _mm_bf16(qb, vposT_ref[...]) + qn2 + vn2

    iota_b = jax.lax.broadcasted_iota(jnp.int32, (_BN, _N), 1).astype(
        jnp.float32)
    masks = []
    p0s, p1s, p2s = [], [], []
    for _ in range(_K):
        m = jnp.min(e, axis=1, keepdims=True)
        cand = jnp.where(e == m, iota_b, float(_N))
        amin = jnp.min(cand, axis=1, keepdims=True)
        msk = iota_b == amin
        mf = msk.astype(jnp.float32)
        masks.append(mf)
        p0s.append(jnp.sum(mf * v0, axis=1, keepdims=True))
        p1s.append(jnp.sum(mf * v1, axis=1, keepdims=True))
        p2s.append(jnp.sum(mf * v2, axis=1, keepdims=True))
        e = jnp.where(msk, _BIG, e)

    G = jnp.concatenate(masks, axis=0).astype(_BF)

    def _minmax(ps):
        lo, hi = ps[0], ps[0]
        for p in ps[1:]:
            lo = jnp.minimum(lo, p)
            hi = jnp.maximum(hi, p)
        return (hi - lo) * 0.5

    sc0, sc1, sc2 = _minmax(p0s), _minmax(p1s), _minmax(p2s)
    lp = jnp.concatenate(
        [jnp.concatenate(ps, axis=0) for ps in (p0s, p1s, p2s)], axis=1)
    sct = jnp.concatenate(
        [jnp.concatenate([s] * _K, axis=0) for s in (sc0, sc1, sc2)], axis=1)

    offl = _gather3(G, vh_ref, vm_ref, vl_ref)
    h2b = h2_ref[pl.ds(i * _BN, _BN), :]
    h = _mm_bf16(offl, W1a_ref[...]) + jnp.concatenate([h2b] * _K, axis=0)

    mean = jnp.mean(h, axis=1, keepdims=True)
    hc = h - mean
    var = jnp.mean(hc * hc, axis=1, keepdims=True)
    hn = hc / jnp.sqrt(var + 1e-5) * lng_ref[...] + lnb_ref[...]
    ge = 0.5 * hn * (1.0 + jax.lax.erf(hn * _INV_SQRT2))
    offs = jnp.tanh(_mm_bf16(ge, W2_ref[...]))

    shift = lp + offs * sct
    s0 = shift[:, 0:1]
    s1 = shift[:, 1:2]
    s2 = shift[:, 2:3]
    sn2 = s0 * s0 + s1 * s1 + s2 * s2

    e2 = -2.0 * _mm_bf16(shift, vposT_ref[...]) + sn2 + vn2

    iota_g = jax.lax.broadcasted_iota(jnp.int32, (_BNK, _N), 1).astype(
        jnp.float32)
    msks = []
    rs = []
    for _ in range(3):
        m = jnp.min(e2, axis=1, keepdims=True)
        cand = jnp.where(e2 == m, iota_g, float(_N))
        amin = jnp.min(cand, axis=1, keepdims=True)
        msk = iota_g == amin
        dist = jnp.sqrt(jnp.clip(m, 1e-12, None))
        rs.append(1.0 / (dist + 1e-8))
        msks.append(msk)
        e2 = jnp.where(msk, _BIG, e2)

    R = (rs[0] + rs[1]) + rs[2]
    w = [r / R for r in rs]
    z = jnp.zeros((_BNK, _N), jnp.float32)
    wh = [x.astype(_BF).astype(jnp.float32) for x in w]
    wl = [x - y for x, y in zip(w, wh)]
    W3h = jnp.where(msks[0], wh[0],
                    jnp.where(msks[1], wh[1],
                              jnp.where(msks[2], wh[2], z))).astype(_BF)
    W3l = jnp.where(msks[0], wl[0],
                    jnp.where(msks[1], wl[1],
                              jnp.where(msks[2], wl[2], z))).astype(_BF)

    interp = (_mm_bf(W3h, qh_ref[...]) + _mm_bf(W3h, qm_ref[...])
              + _mm_bf(W3l, qh_ref[...]))
    qblk = q_ref[pl.ds(i * _BN, _BN), :]
    f = interp - jnp.concatenate([qblk] * _K, axis=0)
    t2b = t2_ref[pl.ds(i * _BN, _BN), :]
    o = _mm_bf16(f, Wka_ref[...]) + jnp.concatenate([t2b] * _K, axis=0)
    o = jnp.where(o >= 0, o, 0.2 * o)

    acc = o[0:_BN, :]
    for k in range(1, _K):
        acc = jnp.maximum(acc, o[k * _BN:(k + 1) * _BN, :])
    out_ref[...] = acc


def kernel(q, q_pos, Wv, bv, W1, b1, ln_g, ln_b, W2, Wk, bk):
    B, N, C = q.shape
    q2 = q[0]
    qpos = q_pos[0]
    vposT = qpos.T
    ph = qpos.astype(_BF)
    r1 = qpos - ph.astype(jnp.float32)
    pm = r1.astype(_BF)
    pl_ = (r1 - pm.astype(jnp.float32)).astype(_BF)
    W1a, W1b = W1[:C], W1[C:]
    Wka = Wk[:C]
    Wkb = Wk[C:]

    full = lambda shape: pl.BlockSpec(shape, lambda i: (0,) * len(shape))
    out = pl.pallas_call(
        _body,
        grid=(_NB,),
        in_specs=[
            full((_N, _C)),
            full((_N, 3)),
            full((3, _N)),
            full((_N, 3)),
            full((_N, 3)),
            full((_N, 3)),
            full((_C, _C)),
            full((1, _C)),
            full((_C, _C)),
            full((_C, _C)),
            full((1, _C)),
            full((1, _C)),
            full((1, _C)),
            full((_C, 3)),
            full((_C, _C)),
            full((_C, _C)),
            full((1, _C)),
        ],
        out_specs=pl.BlockSpec((_BN, _C), lambda i: (i, 0)),
        out_shape=jax.ShapeDtypeStruct((_N, _C), jnp.float32),
        scratch_shapes=[
            pltpu.VMEM((_N, _C), jnp.bfloat16),
            pltpu.VMEM((_N, _C), jnp.bfloat16),
            pltpu.VMEM((_N, _C), jnp.bfloat16),
            pltpu.VMEM((_N, _C), jnp.bfloat16),
            pltpu.VMEM((_N, _C), jnp.bfloat16),
            pltpu.VMEM((_N, _C), jnp.float32),
            pltpu.VMEM((_N, _C), jnp.float32),
        ],
        compiler_params=pltpu.CompilerParams(
            dimension_semantics=("arbitrary",)),
    )(q2, qpos, vposT, ph, pm, pl_, Wv, bv[None, :], W1a, W1b, b1[None, :],
      ln_g[None, :], ln_b[None, :], W2, Wka, Wkb, bk[None, :])
    return out[None]


# speedup vs baseline: 1.9847x; 1.0604x over previous
import jax
import jax.numpy as jnp
from jax.experimental import pallas as pl
from jax.experimental.pallas import tpu as pltpu

_N = 2048
_C = 256
_K = 10
_BN = 128
_NB = _N // _BN
_BNK = _BN * _K
_BIG = 1e30
_INV_SQRT2 = 0.7071067811865476
_BF = jnp.bfloat16


def _mm_bf16(a, b):
    return jnp.dot(a.astype(_BF), b.astype(_BF),
                   preferred_element_type=jnp.float32)


def _mm_bf(a_bf, b_bf):
    return jnp.dot(a_bf, b_bf, preferred_element_type=jnp.float32)


def _split3(x):
    hi = x.astype(_BF)
    r1 = x - hi.astype(jnp.float32)
    mid = r1.astype(_BF)
    r2 = r1 - mid.astype(jnp.float32)
    return hi, mid, r2.astype(_BF)


def _gather3(g_bf, h_ref, m_ref, l_ref):
    return (_mm_bf(g_bf, h_ref[...]) + _mm_bf(g_bf, m_ref[...])
            + _mm_bf(g_bf, l_ref[...]))


def _body(q_ref, qpos_ref, vposT_ref, ph_ref, pm_ref, pl_ref,
          Wv_ref, bv_ref, W1a_ref, W1b_ref,
          b1_ref, lng_ref, lnb_ref, W2_ref, Wka_ref, Wkb_ref, bk_ref,
          out_ref, vh_ref, vm_ref, vl_ref, qh_ref, qm_ref,
          h2_ref, t2_ref):
    i = pl.program_id(0)

    @pl.when(i == 0)
    def _init():
        qq = q_ref[...]
        voff = _mm_bf16(qq, Wv_ref[...]) + bv_ref[...]
        vh_ref[...], vm_ref[...], vl_ref[...] = _split3(voff)
        qh, qm, _ = _split3(qq)
        qh_ref[...] = qh
        qm_ref[...] = qm
        h2_ref[...] = _mm_bf16(qq, W1b_ref[...]) + b1_ref[...]
        t2_ref[...] = _mm_bf16(qq, Wkb_ref[...]) + bk_ref[...]

    v0 = vposT_ref[0:1, :]
    v1 = vposT_ref[1:2, :]
    v2 = vposT_ref[2:3, :]
    vn2 = v0 * v0 + v1 * v1 + v2 * v2

    qb = qpos_ref[pl.ds(i * _BN, _BN), :]

    q0 = qb[:, 0:1]
    q1 = qb[:, 1:2]
    q2 = qb[:, 2:3]
    qn2 = q0 * q0 + q1 * q1 + q2 * q2
    e = -2.0 * _mm_bf16(qb, vposT_ref[...]) + qn2 + vn2

    iota_b = jax.lax.broadcasted_iota(jnp.int32, (_BN, _N), 1).astype(
        jnp.float32)
    masks = []
    p0s, p1s, p2s = [], [], []
    for _ in range(_K):
        m = jnp.min(e, axis=1, keepdims=True)
        cand = jnp.where(e == m, iota_b, float(_N))
        amin = jnp.min(cand, axis=1, keepdims=True)
        msk = iota_b == amin
        mf = msk.astype(jnp.float32)
        masks.append(mf)
        p0s.append(jnp.sum(mf * v0, axis=1, keepdims=True))
        p1s.append(jnp.sum(mf * v1, axis=1, keepdims=True))
        p2s.append(jnp.sum(mf * v2, axis=1, keepdims=True))
        e = jnp.where(msk, _BIG, e)

    G = jnp.concatenate(masks, axis=0).astype(_BF)

    def _minmax(ps):
        lo, hi = ps[0], ps[0]
        for p in ps[1:]:
            lo = jnp.minimum(lo, p)
            hi = jnp.maximum(hi, p)
        return (hi - lo) * 0.5

    sc0, sc1, sc2 = _minmax(p0s), _minmax(p1s), _minmax(p2s)
    lp = jnp.concatenate(
        [jnp.concatenate(ps, axis=0) for ps in (p0s, p1s, p2s)], axis=1)
    sct = jnp.concatenate(
        [jnp.concatenate([s] * _K, axis=0) for s in (sc0, sc1, sc2)], axis=1)

    offl = _gather3(G, vh_ref, vm_ref, vl_ref)
    h2b = h2_ref[pl.ds(i * _BN, _BN), :]
    h = _mm_bf16(offl, W1a_ref[...]) + jnp.concatenate([h2b] * _K, axis=0)

    mean = jnp.mean(h, axis=1, keepdims=True)
    hc = h - mean
    var = jnp.mean(hc * hc, axis=1, keepdims=True)
    hn = hc / jnp.sqrt(var + 1e-5) * lng_ref[...] + lnb_ref[...]
    ge = 0.5 * hn * (1.0 + jax.lax.erf(hn * _INV_SQRT2))
    offs = jnp.tanh(_mm_bf16(ge, W2_ref[...]))

    shift = lp + offs * sct
    s0 = shift[:, 0:1]
    s1 = shift[:, 1:2]
    s2 = shift[:, 2:3]
    sn2 = s0 * s0 + s1 * s1 + s2 * s2

    e2 = -2.0 * _mm_bf16(shift, vposT_ref[...]) + sn2 + vn2

    iota_g = jax.lax.broadcasted_iota(jnp.int32, (_BNK, _N), 1).astype(
        jnp.float32)
    msks = []
    rs = []
    for _ in range(3):
        m = jnp.min(e2, axis=1, keepdims=True)
        cand = jnp.where(e2 == m, iota_g, float(_N))
        amin = jnp.min(cand, axis=1, keepdims=True)
        msk = iota_g == amin
        dist = jnp.sqrt(jnp.clip(m, 1e-12, None))
        rs.append(1.0 / (dist + 1e-8))
        msks.append(msk)
        e2 = jnp.where(msk, _BIG, e2)

    R = (rs[0] + rs[1]) + rs[2]
    w = [r / R for r in rs]
    z = jnp.zeros((_BNK, _N), jnp.float32)
    wh = [x.astype(_BF).astype(jnp.float32) for x in w]
    wl = [x - y for x, y in zip(w, wh)]
    W3h = jnp.where(msks[0], wh[0],
                    jnp.where(msks[1], wh[1],
                              jnp.where(msks[2], wh[2], z))).astype(_BF)
    W3l = jnp.where(msks[0], wl[0],
                    jnp.where(msks[1], wl[1],
                              jnp.where(msks[2], wl[2], z))).astype(_BF)

    interp = (_mm_bf(W3h, qh_ref[...]) + _mm_bf(W3h, qm_ref[...])
              + _mm_bf(W3l, qh_ref[...]))
    qblk = q_ref[pl.ds(i * _BN, _BN), :]
    f = interp - jnp.concatenate([qblk] * _K, axis=0)
    t2b = t2_ref[pl.ds(i * _BN, _BN), :]
    o = _mm_bf16(f, Wka_ref[...]) + jnp.concatenate([t2b] * _K, axis=0)
    o = jnp.where(o >= 0, o, 0.2 * o)

    acc = o[0:_BN, :]
    for k in range(1, _K):
        acc = jnp.maximum(acc, o[k * _BN:(k + 1) * _BN, :])
    out_ref[...] = acc


def kernel(q, q_pos, Wv, bv, W1, b1, ln_g, ln_b, W2, Wk, bk):
    B, N, C = q.shape
    q2 = q[0]
    qpos = q_pos[0]
    vposT = qpos.T
    ph = qpos.astype(_BF)
    r1 = qpos - ph.astype(jnp.float32)
    pm = r1.astype(_BF)
    pl_ = (r1 - pm.astype(jnp.float32)).astype(_BF)
    W1a, W1b = W1[:C], W1[C:]
    Wka = Wk[:C]
    Wkb = Wk[C:]

    full = lambda shape: pl.BlockSpec(shape, lambda i: (0,) * len(shape))
    out = pl.pallas_call(
        _body,
        grid=(_NB,),
        in_specs=[
            full((_N, _C)),
            full((_N, 3)),
            full((3, _N)),
            full((_N, 3)),
            full((_N, 3)),
            full((_N, 3)),
            full((_C, _C)),
            full((1, _C)),
            full((_C, _C)),
            full((_C, _C)),
            full((1, _C)),
            full((1, _C)),
            full((1, _C)),
            full((_C, 3)),
            full((_C, _C)),
            full((_C, _C)),
            full((1, _C)),
        ],
        out_specs=pl.BlockSpec((_BN, _C), lambda i: (i, 0)),
        out_shape=jax.ShapeDtypeStruct((_N, _C), jnp.float32),
        scratch_shapes=[
            pltpu.VMEM((_N, _C), jnp.bfloat16),
            pltpu.VMEM((_N, _C), jnp.bfloat16),
            pltpu.VMEM((_N, _C), jnp.bfloat16),
            pltpu.VMEM((_N, _C), jnp.bfloat16),
            pltpu.VMEM((_N, _C), jnp.bfloat16),
            pltpu.VMEM((_N, _C), jnp.float32),
            pltpu.VMEM((_N, _C), jnp.float32),
        ],
        compiler_params=pltpu.CompilerParams(
            dimension_semantics=("arbitrary",)),
    )(q2, qpos, vposT, ph, pm, pl_, Wv, bv[None, :], W1a, W1b, b1[None, :],
      ln_g[None, :], ln_b[None, :], W2, Wka, Wkb, bk[None, :])
    return out[None]
